# Initial kernel scaffold; baseline (speedup 1.0000x reference)
import jax
import jax.numpy as jnp
from jax.experimental import pallas as pl


def kernel(xs, bottle_neck, lang_w, lang_b, gate_w, w1, b1, w2, b2):
    raise NotImplementedError("write your pallas kernel here")


# speedup vs baseline: 1.2307x; 1.2307x over previous
import functools

import jax
import jax.numpy as jnp
from jax import lax
from jax.experimental import pallas as pl


def _routing_body(x_ref, bn_ref, lw_ref, lb_ref, gw_ref, w8_ref):
    x = x_ref[...]
    bn = bn_ref[...]
    lw = lw_ref[...]
    lb = lb_ref[...]
    gw = gw_ref[...]

    rl = lax.dot_general(bn, lw, (((1,), (1,)), ((), ())),
                         preferred_element_type=jnp.float32) + lb
    gate = lax.dot_general(x, gw, (((1,), (1,)), ((), ())),
                           preferred_element_type=jnp.float32)

    g = rl[:, 1:2] > rl[:, 0:1]
    logits4 = jnp.where(g, gate[:, 4:8], gate[:, 0:4])

    iota4 = lax.broadcasted_iota(jnp.int32, logits4.shape, 1)
    m1 = jnp.max(logits4, axis=1, keepdims=True)
    e1 = jnp.min(jnp.where(logits4 == m1, iota4, 9), axis=1, keepdims=True)
    lmask = jnp.where(iota4 == e1, -1e30, logits4)
    m2 = jnp.max(lmask, axis=1, keepdims=True)
    e2 = jnp.min(jnp.where(lmask == m2, iota4, 9), axis=1, keepdims=True)

    r = jnp.exp(m2 - m1)
    p1 = 1.0 / (1.0 + r)
    p2 = r * p1

    goff = jnp.where(g, 4, 0)
    c1 = goff + e1
    c2 = goff + e2
    iota8 = lax.broadcasted_iota(jnp.int32, (x.shape[0], 8), 1)
    w8 = p1 * (iota8 == c1).astype(jnp.float32) + p2 * (iota8 == c2).astype(jnp.float32)
    w8_ref[...] = w8


def _ffn_body(x_ref, w8_ref, w1_ref, b1_ref, w2_ref, b2_ref, out_ref):
    e = pl.program_id(1)

    @pl.when(e == 0)
    def _init():
        out_ref[...] = jnp.zeros_like(out_ref)

    x = x_ref[...]
    w1 = w1_ref[0]
    w2 = w2_ref[0]
    h = lax.dot_general(x, w1, (((1,), (1,)), ((), ())),
                        preferred_element_type=jnp.float32) + b1_ref[0]
    h = jnp.maximum(h, 0.0)
    y = lax.dot_general(h, w2, (((1,), (1,)), ((), ())),
                        preferred_element_type=jnp.float32) + b2_ref[0]

    iota8 = lax.broadcasted_iota(jnp.int32, w8_ref.shape, 1)
    wcol = jnp.sum(w8_ref[...] * (iota8 == e).astype(jnp.float32), axis=1,
                   keepdims=True)
    out_ref[...] += wcol * y


@jax.jit
def _forward(x, bn, lang_w, lang_b, gate_w8, w1, b1, w2, b2):
    N, D = x.shape
    F = w1.shape[1]
    w8 = pl.pallas_call(
        _routing_body,
        out_shape=jax.ShapeDtypeStruct((N, 8), jnp.float32),
    )(x, bn, lang_w, lang_b.reshape(1, 2), gate_w8)

    BT = 512
    nt = N // BT
    out = pl.pallas_call(
        _ffn_body,
        grid=(nt, 8),
        in_specs=[
            pl.BlockSpec((BT, D), lambda t, e: (t, 0)),
            pl.BlockSpec((BT, 8), lambda t, e: (t, 0)),
            pl.BlockSpec((1, F, D), lambda t, e: (e, 0, 0)),
            pl.BlockSpec((1, 1, F), lambda t, e: (e, 0, 0)),
            pl.BlockSpec((1, D, F), lambda t, e: (e, 0, 0)),
            pl.BlockSpec((1, 1, D), lambda t, e: (e, 0, 0)),
        ],
        out_specs=pl.BlockSpec((BT, D), lambda t, e: (t, 0)),
        out_shape=jax.ShapeDtypeStruct((N, D), jnp.float32),
    )(x, w8, w1, b1, w2, b2)
    return out


def kernel(xs, bottle_neck, lang_w, lang_b, gate_w, w1, b1, w2, b2):
    B, L, D = xs.shape
    N = B * L
    F = w1.shape[2]
    x = xs.reshape(N, D)
    bn = bottle_neck.reshape(N, D)
    gw8 = gate_w.reshape(8, D)
    out = _forward(x, bn, lang_w, lang_b, gw8,
                   w1.reshape(8, F, D), b1.reshape(8, 1, F),
                   w2.reshape(8, D, F), b2.reshape(8, 1, D))
    return out.reshape(B, L, D)
